# Initial kernel scaffold; baseline (speedup 1.0000x reference)
import jax
import jax.numpy as jnp
from jax.experimental import pallas as pl


def kernel(x, y, table):
    raise NotImplementedError("write your pallas kernel here")


# speedup vs baseline: 1.3321x; 1.3321x over previous
import functools
import jax
import jax.numpy as jnp
from jax import lax
from jax.experimental import pallas as pl
from jax.experimental.pallas import tpu as pltpu
from jax.experimental.pallas import tpu_sc as plsc

VOCAB = 1000
N_TOK = 1024 * 50
LSE_PAD = 1024

_info = plsc.get_sparse_core_info()
NC, NS = _info.num_cores, _info.num_subcores
NW = NC * NS
TOK_W = N_TOK // NW
CHUNK = 32
NCHUNK = TOK_W // CHUNK


def _lse_body(tab_ref, out_ref):
    t = tab_ref[...]
    m = jnp.max(t, axis=1, keepdims=True)
    s = jnp.sum(jnp.exp(t - m), axis=1, keepdims=True)
    lse = m[:, 0] + jnp.log(s[:, 0])
    out_ref[...] = jnp.concatenate(
        [lse, jnp.zeros((LSE_PAD - VOCAB,), jnp.float32)])[:, None]


_lse_call = pl.pallas_call(
    _lse_body,
    out_shape=jax.ShapeDtypeStruct((LSE_PAD, 1), jnp.float32),
)


def _finalize_body(part_ref, out_ref):
    out_ref[...] = jnp.sum(part_ref[...]).reshape(1, 1) * (1.0 / N_TOK)


_finalize_call = pl.pallas_call(
    _finalize_body,
    out_shape=jax.ShapeDtypeStruct((1, 1), jnp.float32),
)


_sc_mesh = plsc.VectorSubcoreMesh(core_axis_name="c", subcore_axis_name="s")


@functools.partial(
    pl.kernel,
    mesh=_sc_mesh,
    compiler_params=pltpu.CompilerParams(use_tc_tiling_on_sc=False),
    out_type=[
        jax.ShapeDtypeStruct((N_TOK, VOCAB), jnp.float32),
        jax.ShapeDtypeStruct((NW, 16), jnp.float32),
    ],
    scratch_types=[
        pltpu.VMEM((NCHUNK, CHUNK), jnp.int32),
        pltpu.VMEM((NCHUNK, CHUNK), jnp.int32),
        pltpu.VMEM((NCHUNK, CHUNK), jnp.int32),
        pltpu.VMEM((CHUNK,), jnp.float32),
        pltpu.VMEM((CHUNK,), jnp.float32),
        pltpu.VMEM((CHUNK, VOCAB), jnp.float32),
        pltpu.VMEM((16,), jnp.float32),
        pltpu.SemaphoreType.DMA,
        pltpu.SemaphoreType.DMA,
    ],
)
def _sc_gather(table_hbm, tabflat_hbm, x3_hbm, y3_hbm, lse_hbm,
               logits_hbm, part_hbm,
               x_v, y_v, fidx_v, lsev_v, tgt_v, rows_v, acc_v, gsem, vsem):
    cid = lax.axis_index("c")
    sid = lax.axis_index("s")
    wid = sid * NC + cid
    base = wid * TOK_W

    pltpu.sync_copy(x3_hbm.at[wid], x_v)
    pltpu.sync_copy(y3_hbm.at[wid], y_v)

    def chunk_body(c, acc):
        for g in range(CHUNK // 16):
            sl = pl.ds(g * 16, 16)
            fidx_v[c, sl] = x_v[c, sl] * LSE_PAD + y_v[c, sl]
        rows_cp = pltpu.async_copy(table_hbm.at[x_v.at[c]], rows_v, gsem)
        lse_cp = pltpu.async_copy(lse_hbm.at[x_v.at[c]], lsev_v, vsem)
        tgt_cp = pltpu.async_copy(tabflat_hbm.at[fidx_v.at[c]], tgt_v, vsem)
        lse_cp.wait()
        tgt_cp.wait()
        for g in range(CHUNK // 16):
            sl = pl.ds(g * 16, 16)
            acc = acc + (lsev_v[sl] - tgt_v[sl])
        rows_cp.wait()
        pltpu.sync_copy(rows_v, logits_hbm.at[pl.ds(base + c * CHUNK, CHUNK)])
        return acc

    acc = lax.fori_loop(0, NCHUNK, chunk_body, jnp.zeros((16,), jnp.float32))
    acc_v[...] = acc
    pltpu.sync_copy(acc_v, part_hbm.at[wid])


def kernel(x, y, table):
    B, S = x.shape
    x3 = x.astype(jnp.int32).reshape(NW, NCHUNK, CHUNK)
    y3 = y.astype(jnp.int32).reshape(NW, NCHUNK, CHUNK)
    lse = _lse_call(table).reshape(LSE_PAD)
    tabflat = jnp.pad(table, ((0, 0), (0, LSE_PAD - VOCAB))).reshape(-1)
    logits_flat, partials = _sc_gather(table, tabflat, x3, y3, lse)
    loss = _finalize_call(partials)[0, 0]
    return (logits_flat.reshape(B, S, VOCAB), loss)
